# Initial kernel scaffold; baseline (speedup 1.0000x reference)
import jax
import jax.numpy as jnp
from jax.experimental import pallas as pl


def kernel(x, edge_index, edge_vals, W1, W2, W3):
    raise NotImplementedError("write your pallas kernel here")


# speedup vs baseline: 1.0819x; 1.0819x over previous
import functools

import jax
import jax.numpy as jnp
from jax.experimental import pallas as pl

N = 10000
E = 320000


def _linear_act_body(x_ref, w_ref, o_ref, *, slope):
    y = jnp.dot(x_ref[...], w_ref[...], preferred_element_type=jnp.float32)
    if slope is not None:
        y = jnp.where(y > 0, y, slope * y)
    o_ref[...] = y


def _linear(x, w, slope, m_tile=2000):
    m, k = x.shape
    n = w.shape[1]
    grid = (m // m_tile,)
    return pl.pallas_call(
        functools.partial(_linear_act_body, slope=slope),
        grid=grid,
        in_specs=[
            pl.BlockSpec((m_tile, k), lambda i: (i, 0)),
            pl.BlockSpec((k, n), lambda i: (0, 0)),
        ],
        out_specs=pl.BlockSpec((m_tile, n), lambda i: (i, 0)),
        out_shape=jax.ShapeDtypeStruct((m, n), jnp.float32),
    )(x, w)


def _decoder_body(zl_ref, zr_ref, o_ref):
    logits = jax.lax.dot_general(
        zl_ref[...], zr_ref[...],
        (((1,), (1,)), ((), ())),
        preferred_element_type=jnp.float32,
    )
    o_ref[...] = jax.nn.sigmoid(logits)


def _decoder(z, m_tile=1000, n_tile=2048):
    m = z.shape[0]
    grid = (m // m_tile, pl.cdiv(m, n_tile))
    return pl.pallas_call(
        _decoder_body,
        grid=grid,
        in_specs=[
            pl.BlockSpec((m_tile, z.shape[1]), lambda i, j: (i, 0)),
            pl.BlockSpec((n_tile, z.shape[1]), lambda i, j: (j, 0)),
        ],
        out_specs=pl.BlockSpec((m_tile, n_tile), lambda i, j: (i, j)),
        out_shape=jax.ShapeDtypeStruct((m, m), jnp.float32),
    )(z, z)


def _spmm(edge_index, edge_vals, dense):
    rows = edge_index[0]
    cols = edge_index[1]
    gathered = jnp.take(dense, cols, axis=0) * edge_vals[:, None]
    return jax.ops.segment_sum(gathered, rows, num_segments=N)


def kernel(x, edge_index, edge_vals, W1, W2, W3):
    s1 = _linear(x, W1, 0.2)
    z = _spmm(edge_index, edge_vals, s1)
    s2 = _linear(z, W2, 0.2)
    z = _spmm(edge_index, edge_vals, s2)
    s3 = _linear(z, W3, None)
    z_igae = _spmm(edge_index, edge_vals, s3)
    z_igae_adj = _decoder(z_igae)
    return (z_igae, z_igae_adj)
